# baseline (speedup 1.0000x reference)
import functools

import jax
import jax.numpy as jnp
from jax import lax
from jax.experimental import pallas as pl
from jax.experimental.pallas import tpu as pltpu
from jax.experimental.pallas import tpu_sc as plsc

EMB_D = 64
K = 1024
TPB = 1024
NB = 16
N_TOK = NB * TPB
COMMIT = 0.25


def _vq_tc_body(xc_ref, w_ref,
                idx_ref, q_ref, hist_ref, counts_ref, loss_ref, perp_ref,
                acc_counts, acc_loss):
    b = pl.program_id(0)
    xc = xc_ref[0]
    w = w_ref[...]
    xsq = jnp.sum(xc * xc, axis=0, keepdims=True)
    wsq = jnp.sum(w * w, axis=1, keepdims=True)

    xw = jnp.dot(w, xc, preferred_element_type=jnp.float32)
    dist = (xsq + wsq) - 2.0 * xw

    minv = jnp.min(dist, axis=0, keepdims=True)
    sub = lax.broadcasted_iota(jnp.int32, (K, TPB), 0)
    idx = jnp.min(jnp.where(dist == minv, sub, K), axis=0, keepdims=True)
    idx_ref[pl.ds(b, 1), :] = idx

    e = jnp.exp(minv - dist)
    s = jnp.sum(e, axis=0, keepdims=True)
    es = e * (1.0 / s)
    hist_ref[0] = jnp.sum(es, axis=1, keepdims=True)

    onehot = (sub == idx).astype(jnp.float32)
    counts_col = jnp.sum(onehot, axis=1, keepdims=True)
    counts_ref[0] = counts_col

    qc = lax.dot_general(w, onehot, (((0,), (0,)), ((), ())),
                         preferred_element_type=jnp.float32)
    q_ref[0] = qc.reshape(EMB_D, 32, 32)

    lp = jnp.sum(minv, axis=1, keepdims=True)

    @pl.when(b == 0)
    def _():
        acc_counts[...] = counts_col
        acc_loss[...] = lp

    @pl.when(b > 0)
    def _():
        acc_counts[...] += counts_col
        acc_loss[...] += lp

    @pl.when(b == NB - 1)
    def _():
        avg = acc_counts[...] * (1.0 / N_TOK)
        ent = jnp.sum(avg * jnp.log(avg + 1e-10), axis=0, keepdims=True)
        perp_ref[0] = jnp.exp(-ent)[0, 0]
        loss_ref[0] = acc_loss[0, 0] * ((1.0 + COMMIT) / (N_TOK * EMB_D))


def _build_tc(interpret=False):
    return pl.pallas_call(
        _vq_tc_body,
        grid=(NB,),
        in_specs=[
            pl.BlockSpec((1, EMB_D, TPB), lambda b: (b, 0, 0)),
            pl.BlockSpec((K, EMB_D), lambda b: (0, 0)),
        ],
        out_specs=[
            pl.BlockSpec((NB, TPB), lambda b: (0, 0)),
            pl.BlockSpec((1, EMB_D, 32, 32), lambda b: (b, 0, 0, 0)),
            pl.BlockSpec((1, K, 1), lambda b: (b, 0, 0)),
            pl.BlockSpec((1, K, 1), lambda b: (b, 0, 0)),
            pl.BlockSpec(memory_space=pltpu.SMEM),
            pl.BlockSpec(memory_space=pltpu.SMEM),
        ],
        out_shape=[
            jax.ShapeDtypeStruct((NB, TPB), jnp.int32),
            jax.ShapeDtypeStruct((NB, EMB_D, 32, 32), jnp.float32),
            jax.ShapeDtypeStruct((NB, K, 1), jnp.float32),
            jax.ShapeDtypeStruct((NB, K, 1), jnp.float32),
            jax.ShapeDtypeStruct((1,), jnp.float32),
            jax.ShapeDtypeStruct((1,), jnp.float32),
        ],
        scratch_shapes=[
            pltpu.VMEM((K, 1), jnp.float32),
            pltpu.VMEM((1, 1), jnp.float32),
        ],
        interpret=interpret,
    )


def kernel(input, W):
    xc = input.reshape(NB, EMB_D, TPB)
    idx, q, hist_t, counts_t, loss, perp = _build_tc()(xc, W)
    return (q, loss[0], perp[0], idx,
            counts_t.reshape(NB, K), hist_t.reshape(NB, K))


# speedup vs baseline: 1.1767x; 1.1767x over previous
import jax
import jax.numpy as jnp
from jax import lax
from jax.experimental import pallas as pl
from jax.experimental.pallas import tpu as pltpu

EMB_D = 64
K = 1024
TPB = 1024
NB = 16
N_TOK = NB * TPB
COMMIT = 0.25


def _vq_tc_body(xc_ref, w_ref,
                idx_ref, q_ref, hist_ref, counts_ref, loss_ref, perp_ref,
                acc_counts, acc_loss):
    b = pl.program_id(0)
    xc = xc_ref[0]
    w = w_ref[...]
    xsq = jnp.sum(xc * xc, axis=0, keepdims=True)
    wsq = jnp.sum(w * w, axis=1, keepdims=True)

    xw = jnp.dot(w, xc, preferred_element_type=jnp.float32)
    dist = (xsq + wsq) - 2.0 * xw

    minv = jnp.min(dist, axis=0, keepdims=True)
    sub = lax.broadcasted_iota(jnp.int32, (K, TPB), 0)
    idx = jnp.min(jnp.where(dist == minv, sub, K), axis=0, keepdims=True)
    idx_ref[pl.ds(b, 1), :] = idx

    e = jnp.exp(minv - dist)
    s = jnp.sum(e, axis=0, keepdims=True)
    es = e * (1.0 / s)
    hist_ref[0] = jnp.sum(es, axis=1, keepdims=True)

    onehot = (sub == idx).astype(jnp.float32)
    counts_col = jnp.sum(onehot, axis=1, keepdims=True)
    counts_ref[0] = counts_col

    q_ref[0] = lax.dot_general(w, onehot, (((0,), (0,)), ((), ())),
                               preferred_element_type=jnp.float32)

    lp = jnp.sum(minv, axis=1, keepdims=True)

    @pl.when(b == 0)
    def _():
        acc_counts[...] = counts_col
        acc_loss[...] = lp

    @pl.when(b > 0)
    def _():
        acc_counts[...] += counts_col
        acc_loss[...] += lp

    @pl.when(b == NB - 1)
    def _():
        avg = acc_counts[...] * (1.0 / N_TOK)
        ent = jnp.sum(avg * jnp.log(avg + 1e-10), axis=0, keepdims=True)
        perp_ref[0] = jnp.exp(-ent)[0, 0]
        loss_ref[0] = acc_loss[0, 0] * ((1.0 + COMMIT) / (N_TOK * EMB_D))


def _build_tc(interpret=False):
    return pl.pallas_call(
        _vq_tc_body,
        grid=(NB,),
        in_specs=[
            pl.BlockSpec((1, EMB_D, TPB), lambda b: (b, 0, 0)),
            pl.BlockSpec((K, EMB_D), lambda b: (0, 0)),
        ],
        out_specs=[
            pl.BlockSpec((NB, TPB), lambda b: (0, 0)),
            pl.BlockSpec((1, EMB_D, TPB), lambda b: (b, 0, 0)),
            pl.BlockSpec((1, K, 1), lambda b: (b, 0, 0)),
            pl.BlockSpec((1, K, 1), lambda b: (b, 0, 0)),
            pl.BlockSpec(memory_space=pltpu.SMEM),
            pl.BlockSpec(memory_space=pltpu.SMEM),
        ],
        out_shape=[
            jax.ShapeDtypeStruct((NB, TPB), jnp.int32),
            jax.ShapeDtypeStruct((NB, EMB_D, TPB), jnp.float32),
            jax.ShapeDtypeStruct((NB, K, 1), jnp.float32),
            jax.ShapeDtypeStruct((NB, K, 1), jnp.float32),
            jax.ShapeDtypeStruct((1,), jnp.float32),
            jax.ShapeDtypeStruct((1,), jnp.float32),
        ],
        scratch_shapes=[
            pltpu.VMEM((K, 1), jnp.float32),
            pltpu.VMEM((1, 1), jnp.float32),
        ],
        interpret=interpret,
    )


def kernel(input, W):
    xc = input.reshape(NB, EMB_D, TPB)
    idx, q, hist_t, counts_t, loss, perp = _build_tc()(xc, W)
    quantized_out = q.reshape(NB, EMB_D, 32, 32)

    return (quantized_out, loss[0], perp[0], idx,
            counts_t.reshape(NB, K), hist_t.reshape(NB, K))
